# baseline (speedup 1.0000x reference)
import jax
import jax.numpy as jnp
from jax import lax
from jax.experimental import pallas as pl
from jax.experimental.pallas import tpu as pltpu
from jax.experimental.pallas import tpu_sc as plsc

BATCH = 16384
EMBED_DIM = 16
BUCKETS = 100000
ROWS_PER_LINE = 8
LINES = BUCKETS // ROWS_PER_LINE
NUM_CORES = 2
NUM_SUBCORES = 16
NUM_WORKERS = NUM_CORES * NUM_SUBCORES
B_PER_W = BATCH // NUM_WORKERS
LANES = 16
CHUNK = 128
NCHUNK = B_PER_W // CHUNK


def _fm_body(uid_hbm, tid_hbm, utab_hbm, itab_hbm, wb_hbm, out_hbm,
             idx_u_v, idx_t_v, gid_u_v, gid_t_v, out_v, wb_v,
             bu0, bu1, bt0, bt1, su0, su1, st0, st1):
    wid = lax.axis_index("s") * NUM_CORES + lax.axis_index("c")
    base = wid * B_PER_W

    pltpu.sync_copy(uid_hbm.at[pl.ds(base, B_PER_W)], idx_u_v)
    pltpu.sync_copy(tid_hbm.at[pl.ds(base, B_PER_W)], idx_t_v)
    pltpu.sync_copy(wb_hbm, wb_v)

    def gid_body(i, carry):
        s = pl.ds(i * LANES, LANES)
        gid_u_v[s] = lax.shift_right_logical(idx_u_v[s], 3)
        gid_t_v[s] = lax.shift_right_logical(idx_t_v[s], 3)
        return carry

    lax.fori_loop(0, B_PER_W // LANES, gid_body, 0)

    bu = (bu0, bu1)
    bt = (bt0, bt1)
    su = (su0, su1)
    st = (st0, st1)

    def start(c):
        sl = pl.ds(c * CHUNK, CHUNK)
        cu = pltpu.async_copy(utab_hbm.at[gid_u_v.at[sl]], bu[c % 2], su[c % 2])
        ct = pltpu.async_copy(itab_hbm.at[gid_t_v.at[sl]], bt[c % 2], st[c % 2])
        return cu, ct

    lanes = lax.iota(jnp.int32, LANES)
    wv = wb_v[pl.ds(0, LANES)]
    bv = wb_v[pl.ds(LANES, LANES)]

    inflight = start(0)
    for c in range(NCHUNK):
        nxt = start(c + 1) if c + 1 < NCHUNK else None
        inflight[0].wait()
        inflight[1].wait()
        inflight = nxt
        bu_c, bt_c = bu[c % 2], bt[c % 2]

        def grp(g, carry):
            s = pl.ds(c * CHUNK + g * LANES, LANES)
            offu = lax.shift_left(jnp.bitwise_and(idx_u_v[s], 7), 4)
            offt = lax.shift_left(jnp.bitwise_and(idx_t_v[s], 7), 4)
            row = g * LANES + lanes
            acc = jnp.zeros((LANES,), jnp.float32)
            for d in range(EMBED_DIM):
                cu = plsc.load_gather(bu_c, [row, offu + d])
                ct = plsc.load_gather(bt_c, [row, offt + d])
                acc = acc + cu * ct
            z = acc * wv + bv
            y = 1.0 / (1.0 + jnp.exp(-z))
            out_v[s] = y
            return carry

        lax.fori_loop(0, CHUNK // LANES, grp, 0)

    pltpu.sync_copy(out_v, out_hbm.at[pl.ds(base, B_PER_W)])


@jax.jit
def _fm_sc(f_uid, f_tid, utab_lines, itab_lines, wb):
    mesh = plsc.VectorSubcoreMesh(core_axis_name="c", subcore_axis_name="s")
    return pl.kernel(
        _fm_body,
        out_type=jax.ShapeDtypeStruct((BATCH,), jnp.float32),
        mesh=mesh,
        compiler_params=pltpu.CompilerParams(needs_layout_passes=False),
        scratch_types=[
            pltpu.VMEM((B_PER_W,), jnp.int32),
            pltpu.VMEM((B_PER_W,), jnp.int32),
            pltpu.VMEM((B_PER_W,), jnp.int32),
            pltpu.VMEM((B_PER_W,), jnp.int32),
            pltpu.VMEM((B_PER_W,), jnp.float32),
            pltpu.VMEM((8 * LANES,), jnp.float32),
            pltpu.VMEM((CHUNK, 8 * EMBED_DIM), jnp.float32),
            pltpu.VMEM((CHUNK, 8 * EMBED_DIM), jnp.float32),
            pltpu.VMEM((CHUNK, 8 * EMBED_DIM), jnp.float32),
            pltpu.VMEM((CHUNK, 8 * EMBED_DIM), jnp.float32),
            pltpu.SemaphoreType.DMA,
            pltpu.SemaphoreType.DMA,
            pltpu.SemaphoreType.DMA,
            pltpu.SemaphoreType.DMA,
        ],
    )(f_uid, f_tid, utab_lines, itab_lines, wb)


def kernel(f_uid, f_tid, user_table, item_table, W, b):
    utab_lines = user_table.reshape(LINES, ROWS_PER_LINE * EMBED_DIM)
    itab_lines = item_table.reshape(LINES, ROWS_PER_LINE * EMBED_DIM)
    wb = jnp.concatenate([
        jnp.broadcast_to(W.reshape(1), (LANES,)),
        jnp.broadcast_to(b.reshape(1), (LANES,)),
        jnp.zeros((8 * LANES - 2 * LANES,), jnp.float32),
    ])
    y = _fm_sc(f_uid.astype(jnp.int32), f_tid.astype(jnp.int32),
               utab_lines, itab_lines, wb)
    return y.reshape(BATCH, 1)


# speedup vs baseline: 1.8988x; 1.8988x over previous
import jax
import jax.numpy as jnp
from jax import lax
from jax.experimental import pallas as pl
from jax.experimental.pallas import tpu as pltpu
from jax.experimental.pallas import tpu_sc as plsc

BATCH = 16384
EMBED_DIM = 16
BUCKETS = 100000
NUM_CORES = 2
NUM_SUBCORES = 16
NUM_WORKERS = NUM_CORES * NUM_SUBCORES
B_PER_W = BATCH // NUM_WORKERS
LANES = 16
NWORDS = B_PER_W * EMBED_DIM


def _fm_body(uid_hbm, tid_hbm, utab_hbm, itab_hbm, wb_hbm, out_hbm,
             idx_u_v, idx_t_v, fid_u_v, fid_t_v, gu_v, gt_v, out_v, wb_v,
             sem_u, sem_t):
    wid = lax.axis_index("s") * NUM_CORES + lax.axis_index("c")
    base = wid * B_PER_W

    pltpu.sync_copy(uid_hbm.at[pl.ds(base, B_PER_W)], idx_u_v)
    pltpu.sync_copy(tid_hbm.at[pl.ds(base, B_PER_W)], idx_t_v)
    pltpu.sync_copy(wb_hbm, wb_v)

    def fid_body(j, carry):
        iu = idx_u_v[pl.ds(j * LANES, LANES)]
        it = idx_t_v[pl.ds(j * LANES, LANES)]
        for d in range(EMBED_DIM):
            fid_u_v[pl.ds(d * B_PER_W + j * LANES, LANES)] = iu + (d * BUCKETS)
            fid_t_v[pl.ds(d * B_PER_W + j * LANES, LANES)] = it + (d * BUCKETS)
        return carry

    lax.fori_loop(0, B_PER_W // LANES, fid_body, 0)

    cu = pltpu.async_copy(utab_hbm.at[fid_u_v], gu_v, sem_u)
    ct = pltpu.async_copy(itab_hbm.at[fid_t_v], gt_v, sem_t)
    cu.wait()
    ct.wait()

    wv = wb_v[pl.ds(0, LANES)]
    bv = wb_v[pl.ds(LANES, LANES)]

    def grp(j, carry):
        acc = jnp.zeros((LANES,), jnp.float32)
        for d in range(EMBED_DIM):
            s = pl.ds(d * B_PER_W + j * LANES, LANES)
            acc = acc + gu_v[s] * gt_v[s]
        z = acc * wv + bv
        y = 1.0 / (1.0 + jnp.exp(-z))
        out_v[pl.ds(j * LANES, LANES)] = y
        return carry

    lax.fori_loop(0, B_PER_W // LANES, grp, 0)

    pltpu.sync_copy(out_v, out_hbm.at[pl.ds(base, B_PER_W)])


@jax.jit
def _fm_sc(f_uid, f_tid, utab_flat, itab_flat, wb):
    mesh = plsc.VectorSubcoreMesh(core_axis_name="c", subcore_axis_name="s")
    return pl.kernel(
        _fm_body,
        out_type=jax.ShapeDtypeStruct((BATCH,), jnp.float32),
        mesh=mesh,
        compiler_params=pltpu.CompilerParams(needs_layout_passes=False),
        scratch_types=[
            pltpu.VMEM((B_PER_W,), jnp.int32),
            pltpu.VMEM((B_PER_W,), jnp.int32),
            pltpu.VMEM((NWORDS,), jnp.int32),
            pltpu.VMEM((NWORDS,), jnp.int32),
            pltpu.VMEM((NWORDS,), jnp.float32),
            pltpu.VMEM((NWORDS,), jnp.float32),
            pltpu.VMEM((B_PER_W,), jnp.float32),
            pltpu.VMEM((8 * LANES,), jnp.float32),
            pltpu.SemaphoreType.DMA,
            pltpu.SemaphoreType.DMA,
        ],
    )(f_uid, f_tid, utab_flat, itab_flat, wb)


def kernel(f_uid, f_tid, user_table, item_table, W, b):
    utab_flat = user_table.T.reshape(-1)
    itab_flat = item_table.T.reshape(-1)
    wb = jnp.concatenate([
        jnp.broadcast_to(W.reshape(1), (LANES,)),
        jnp.broadcast_to(b.reshape(1), (LANES,)),
        jnp.zeros((8 * LANES - 2 * LANES,), jnp.float32),
    ])
    y = _fm_sc(f_uid.astype(jnp.int32), f_tid.astype(jnp.int32),
               utab_flat, itab_flat, wb)
    return y.reshape(BATCH, 1)
